# baseline (speedup 1.0000x reference)
import functools

import jax
import jax.numpy as jnp
from jax import lax
from jax.experimental import pallas as pl
from jax.experimental.pallas import tpu as pltpu
from jax.experimental.pallas import tpu_sc as plsc

N = 10000
D = 128
E = 320000

NC = 2
NS = 16
NW = NC * NS
CHUNK = 128
CPW = 80
SPC = 81
U2 = 4
E_PAD = NW * CPW * CHUNK
N_PAD = 10240
RPT = N_PAD // NS

_MESH = plsc.VectorSubcoreMesh(core_axis_name="c", subcore_axis_name="s")


def _sc_body(with_cnt, h_hbm, src_hbm, dst_hbm, agg_out, cnt_a, cnt_b,
             shared_agg, shared_cnt, rows, isv, idv, ones_v, zc_v,
             gsem, ssem, dsem):
    cid = lax.axis_index("c")
    sid = lax.axis_index("s")
    wid = sid * NC + cid

    zvec = jnp.zeros((16,), jnp.float32)

    def _zrow(i, _):
        for j in range(8):
            rows[0][i, pl.ds(j * 16, 16)] = zvec
        return 0

    lax.fori_loop(0, CHUNK, _zrow, 0)

    def _zc(i, _):
        zc_v[pl.ds(pl.multiple_of(i * 16, 16), 16)] = zvec
        return 0

    lax.fori_loop(0, RPT // 16, _zc, 0)

    if with_cnt:
        ovec = jnp.ones((16,), jnp.float32)

        def _ones(i, _):
            ones_v[pl.ds(pl.multiple_of(i * 16, 16), 16)] = ovec
            return 0

        lax.fori_loop(0, CHUNK // 16, _ones, 0)

    for u in range(RPT // CHUNK):
        pltpu.sync_copy(rows[0], shared_agg.at[pl.ds(sid * RPT + u * CHUNK, CHUNK)])
    pltpu.sync_copy(zc_v, shared_cnt.at[pl.ds(sid * RPT, RPT)])
    plsc.subcore_barrier()

    base = wid * (SPC * CHUNK)

    def _off(j):
        return pl.multiple_of(base + j * CHUNK, 8)

    def _quad(q, _):
        j0 = q * U2
        ds = [pltpu.async_copy(src_hbm.at[pl.ds(_off(j0 + k), CHUNK)],
                               isv[k], ssem[k]) for k in range(U2)]
        dd = [pltpu.async_copy(dst_hbm.at[pl.ds(_off(j0 + k), CHUNK)],
                               idv[k], dsem[k]) for k in range(U2)]
        gs = [None] * U2
        ds[0].wait()
        gs[0] = pltpu.async_copy(h_hbm.at[isv[0]], rows[0], gsem[0])
        ds[1].wait()
        gs[1] = pltpu.async_copy(h_hbm.at[isv[1]], rows[1], gsem[1])
        for k in range(U2):
            gs[k].wait()
            dd[k].wait()
            pltpu.sync_copy(rows[k & 1], shared_agg.at[idv[k]], add=True)
            if with_cnt:
                pltpu.sync_copy(ones_v, shared_cnt.at[idv[k]], add=True)
            if k + 2 < U2:
                ds[k + 2].wait()
                gs[k + 2] = pltpu.async_copy(h_hbm.at[isv[k + 2]],
                                             rows[k & 1], gsem[k & 1])
        return 0

    lax.fori_loop(0, CPW // U2, _quad, 0)
    plsc.subcore_barrier()

    row0 = sid * RPT
    pltpu.sync_copy(shared_agg.at[pl.ds(row0, RPT)],
                    agg_out.at[pl.ds(cid * N_PAD + row0, RPT)])
    if with_cnt:
        @pl.when(cid == 0)
        def _():
            pltpu.sync_copy(shared_cnt.at[pl.ds(row0, RPT)],
                            cnt_a.at[pl.ds(row0, RPT)])

        @pl.when(cid == 1)
        def _():
            pltpu.sync_copy(shared_cnt.at[pl.ds(row0, RPT)],
                            cnt_b.at[pl.ds(row0, RPT)])


def _make_sc(with_cnt):
    outs = [jax.ShapeDtypeStruct((NC * N_PAD, D), jnp.float32)]
    if with_cnt:
        outs += [jax.ShapeDtypeStruct((N_PAD,), jnp.float32)] * 2
    body = functools.partial(_sc_body, with_cnt)
    if not with_cnt:
        def body(h, srcv, dstv, agg, *rest):
            return _sc_body(False, h, srcv, dstv, agg, None, None, *rest)
    return pl.kernel(
        body,
        out_type=outs,
        mesh=_MESH,
        scratch_types=[
            pltpu.VMEM_SHARED((N_PAD, D), jnp.float32),
            pltpu.VMEM_SHARED((N_PAD,), jnp.float32),
            [pltpu.VMEM((CHUNK, D), jnp.float32) for _ in range(2)],
            [pltpu.VMEM((CHUNK,), jnp.int32) for _ in range(U2)],
            [pltpu.VMEM((CHUNK,), jnp.int32) for _ in range(U2)],
            pltpu.VMEM((CHUNK,), jnp.float32),
            pltpu.VMEM((RPT,), jnp.float32),
            [pltpu.SemaphoreType.DMA for _ in range(2)],
            [pltpu.SemaphoreType.DMA for _ in range(U2)],
            [pltpu.SemaphoreType.DMA for _ in range(U2)],
        ],
    )


_sc_agg_cnt = _make_sc(True)
_sc_agg = _make_sc(False)

BN = 1024


def _tc_body(relu, agg0_ref, agg1_ref, ca_ref, cb_ref, h_ref, wn_ref, ws_ref,
             b_ref, out_ref):
    cnt = ca_ref[...] + cb_ref[...]
    inv = 1.0 / jnp.maximum(cnt, 1.0)
    agg = (agg0_ref[0] + agg1_ref[0]) * inv[:, None]
    acc = jnp.dot(agg, wn_ref[...], preferred_element_type=jnp.float32)
    acc += jnp.dot(h_ref[...], ws_ref[...], preferred_element_type=jnp.float32)
    acc += b_ref[...][None, :]
    if relu:
        acc = jnp.maximum(acc, 0.0)
    out_ref[...] = acc


def _make_tc(relu):
    grid = N_PAD // BN
    return pl.pallas_call(
        functools.partial(_tc_body, relu),
        grid=(grid,),
        in_specs=[
            pl.BlockSpec((1, BN, D), lambda i: (0, i, 0)),
            pl.BlockSpec((1, BN, D), lambda i: (1, i, 0)),
            pl.BlockSpec((BN,), lambda i: (i,)),
            pl.BlockSpec((BN,), lambda i: (i,)),
            pl.BlockSpec((BN, D), lambda i: (i, 0)),
            pl.BlockSpec((D, D), lambda i: (0, 0)),
            pl.BlockSpec((D, D), lambda i: (0, 0)),
            pl.BlockSpec((D,), lambda i: (0,)),
        ],
        out_specs=pl.BlockSpec((BN, D), lambda i: (i, 0)),
        out_shape=jax.ShapeDtypeStruct((N, D), jnp.float32),
    )


_tc_relu = _make_tc(True)
_tc_lin = _make_tc(False)


def kernel(x, edge_index, W_self_0, W_neigh_0, b_0, W_self_1, W_neigh_1, b_1,
           W_self_2, W_neigh_2, b_2):
    pad = E_PAD - E
    src = jnp.concatenate([edge_index[0], jnp.zeros((pad,), jnp.int32)])
    dst = jnp.concatenate([edge_index[1], jnp.full((pad,), N, jnp.int32)])
    src = jnp.pad(src.reshape(NW, CPW * CHUNK), ((0, 0), (0, CHUNK))).reshape(-1)
    dst = jnp.pad(dst.reshape(NW, CPW * CHUNK), ((0, 0), (0, CHUNK)),
                  constant_values=N).reshape(-1)
    agg_f, cnt_a, cnt_b = _sc_agg_cnt(x, src, dst)
    agg = agg_f.reshape(NC, N_PAD, D)
    h = _tc_relu(agg, agg, cnt_a, cnt_b, x, W_neigh_0, W_self_0, b_0)

    agg = _sc_agg(h, src, dst)[0].reshape(NC, N_PAD, D)
    h = _tc_relu(agg, agg, cnt_a, cnt_b, h, W_neigh_1, W_self_1, b_1)

    agg = _sc_agg(h, src, dst)[0].reshape(NC, N_PAD, D)
    return _tc_lin(agg, agg, cnt_a, cnt_b, h, W_neigh_2, W_self_2, b_2)


# speedup vs baseline: 1.2751x; 1.2751x over previous
import functools

import jax
import jax.numpy as jnp
from jax import lax
from jax.experimental import pallas as pl
from jax.experimental.pallas import tpu as pltpu
from jax.experimental.pallas import tpu_sc as plsc

N = 10000
D = 128
E = 320000

NC = 2
NS = 16
NW = NC * NS
CHUNK = 128
CPW = -(-E // (NW * CHUNK))
E_PAD = NW * CPW * CHUNK
N_PAD = 10240
RPT = N_PAD // NS

_MESH = plsc.VectorSubcoreMesh(core_axis_name="c", subcore_axis_name="s")


def _sc_body(with_cnt, h_hbm, src_hbm, dst_hbm, agg_out, cnt_a, cnt_b,
             shared_agg, shared_cnt, rows_v, idx_s, idx_d, ones_v, zc_v, sem):
    cid = lax.axis_index("c")
    sid = lax.axis_index("s")
    wid = sid * NC + cid

    zvec = jnp.zeros((16,), jnp.float32)

    def _zrow(i, _):
        for j in range(8):
            rows_v[i, pl.ds(j * 16, 16)] = zvec
        return 0

    lax.fori_loop(0, CHUNK, _zrow, 0)

    def _zc(i, _):
        zc_v[pl.ds(pl.multiple_of(i * 16, 16), 16)] = zvec
        return 0

    lax.fori_loop(0, RPT // 16, _zc, 0)

    if with_cnt:
        ovec = jnp.ones((16,), jnp.float32)

        def _ones(i, _):
            ones_v[pl.ds(pl.multiple_of(i * 16, 16), 16)] = ovec
            return 0

        lax.fori_loop(0, CHUNK // 16, _ones, 0)

    for u in range(RPT // CHUNK):
        pltpu.sync_copy(rows_v, shared_agg.at[pl.ds(sid * RPT + u * CHUNK, CHUNK)])
    pltpu.sync_copy(zc_v, shared_cnt.at[pl.ds(sid * RPT, RPT)])
    plsc.subcore_barrier()

    base = wid * (CPW * CHUNK)

    def _edge_chunk(j, _):
        off = pl.multiple_of(base + j * CHUNK, 8)
        pltpu.sync_copy(src_hbm.at[pl.ds(off, CHUNK)], idx_s)
        pltpu.sync_copy(dst_hbm.at[pl.ds(off, CHUNK)], idx_d)
        pltpu.async_copy(h_hbm.at[idx_s], rows_v, sem).wait()
        pltpu.sync_copy(rows_v, shared_agg.at[idx_d], add=True)
        if with_cnt:
            pltpu.sync_copy(ones_v, shared_cnt.at[idx_d], add=True)
        return 0

    lax.fori_loop(0, CPW, _edge_chunk, 0)
    plsc.subcore_barrier()

    row0 = sid * RPT
    pltpu.sync_copy(shared_agg.at[pl.ds(row0, RPT)],
                    agg_out.at[pl.ds(cid * N_PAD + row0, RPT)])
    if with_cnt:
        @pl.when(cid == 0)
        def _():
            pltpu.sync_copy(shared_cnt.at[pl.ds(row0, RPT)],
                            cnt_a.at[pl.ds(row0, RPT)])

        @pl.when(cid == 1)
        def _():
            pltpu.sync_copy(shared_cnt.at[pl.ds(row0, RPT)],
                            cnt_b.at[pl.ds(row0, RPT)])


def _make_sc(with_cnt):
    outs = [jax.ShapeDtypeStruct((NC * N_PAD, D), jnp.float32)]
    if with_cnt:
        outs += [jax.ShapeDtypeStruct((N_PAD,), jnp.float32)] * 2
    body = functools.partial(_sc_body, with_cnt)
    if not with_cnt:
        def body(h, srcv, dstv, agg, *rest):
            return _sc_body(False, h, srcv, dstv, agg, None, None, *rest)
    return pl.kernel(
        body,
        out_type=outs,
        mesh=_MESH,
        scratch_types=[
            pltpu.VMEM_SHARED((N_PAD, D), jnp.float32),
            pltpu.VMEM_SHARED((N_PAD,), jnp.float32),
            pltpu.VMEM((CHUNK, D), jnp.float32),
            pltpu.VMEM((CHUNK,), jnp.int32),
            pltpu.VMEM((CHUNK,), jnp.int32),
            pltpu.VMEM((CHUNK,), jnp.float32),
            pltpu.VMEM((RPT,), jnp.float32),
            pltpu.SemaphoreType.DMA,
        ],
    )


_sc_agg_cnt = _make_sc(True)
_sc_agg = _make_sc(False)

BN = 1024


def _tc_body(relu, agg0_ref, agg1_ref, ca_ref, cb_ref, h_ref, wn_ref, ws_ref,
             b_ref, out_ref):
    cnt = ca_ref[...] + cb_ref[...]
    inv = 1.0 / jnp.maximum(cnt, 1.0)
    agg = (agg0_ref[0] + agg1_ref[0]) * inv[:, None]
    acc = jnp.dot(agg, wn_ref[...], preferred_element_type=jnp.float32)
    acc += jnp.dot(h_ref[...], ws_ref[...], preferred_element_type=jnp.float32)
    acc += b_ref[...][None, :]
    if relu:
        acc = jnp.maximum(acc, 0.0)
    out_ref[...] = acc


def _make_tc(relu):
    grid = N_PAD // BN
    return pl.pallas_call(
        functools.partial(_tc_body, relu),
        grid=(grid,),
        in_specs=[
            pl.BlockSpec((1, BN, D), lambda i: (0, i, 0)),
            pl.BlockSpec((1, BN, D), lambda i: (1, i, 0)),
            pl.BlockSpec((BN,), lambda i: (i,)),
            pl.BlockSpec((BN,), lambda i: (i,)),
            pl.BlockSpec((BN, D), lambda i: (i, 0)),
            pl.BlockSpec((D, D), lambda i: (0, 0)),
            pl.BlockSpec((D, D), lambda i: (0, 0)),
            pl.BlockSpec((D,), lambda i: (0,)),
        ],
        out_specs=pl.BlockSpec((BN, D), lambda i: (i, 0)),
        out_shape=jax.ShapeDtypeStruct((N, D), jnp.float32),
    )


_tc_relu = _make_tc(True)
_tc_lin = _make_tc(False)


def kernel(x, edge_index, W_self_0, W_neigh_0, b_0, W_self_1, W_neigh_1, b_1,
           W_self_2, W_neigh_2, b_2):
    pad = E_PAD - E
    src = jnp.concatenate([edge_index[0], jnp.zeros((pad,), jnp.int32)])
    dst = jnp.concatenate([edge_index[1], jnp.full((pad,), N, jnp.int32)])
    agg_f, cnt_a, cnt_b = _sc_agg_cnt(x, src, dst)
    agg = agg_f.reshape(NC, N_PAD, D)
    h = _tc_relu(agg, agg, cnt_a, cnt_b, x, W_neigh_0, W_self_0, b_0)

    agg = _sc_agg(h, src, dst)[0].reshape(NC, N_PAD, D)
    h = _tc_relu(agg, agg, cnt_a, cnt_b, h, W_neigh_1, W_self_1, b_1)

    agg = _sc_agg(h, src, dst)[0].reshape(NC, N_PAD, D)
    return _tc_lin(agg, agg, cnt_a, cnt_b, h, W_neigh_2, W_self_2, b_2)


# speedup vs baseline: 1.4137x; 1.1087x over previous
import functools

import jax
import jax.numpy as jnp
from jax import lax
from jax.experimental import pallas as pl
from jax.experimental.pallas import tpu as pltpu
from jax.experimental.pallas import tpu_sc as plsc

N = 10000
D = 128
E = 320000

NC = 2
NS = 16
NW = NC * NS
CHUNK = 128
CPW = -(-E // (NW * CHUNK))
CF = 95
CS = 2 * CPW - CF
E_PAD = NW * CPW * CHUNK
N_PAD = 10240
RPT = N_PAD // NS

_MESH = plsc.VectorSubcoreMesh(core_axis_name="c", subcore_axis_name="s")


def _sc_body(with_cnt, h_hbm, src_hbm, dst_hbm, agg_out, cnt_a, cnt_b,
             shared_agg, shared_cnt, rows_v, idx_s, idx_d, ones_v, zc_v, sem):
    cid = lax.axis_index("c")
    sid = lax.axis_index("s")
    wid = sid * NC + cid

    zvec = jnp.zeros((16,), jnp.float32)

    def _zrow(i, _):
        for j in range(8):
            rows_v[i, pl.ds(j * 16, 16)] = zvec
        return 0

    lax.fori_loop(0, CHUNK, _zrow, 0)

    def _zc(i, _):
        zc_v[pl.ds(pl.multiple_of(i * 16, 16), 16)] = zvec
        return 0

    lax.fori_loop(0, RPT // 16, _zc, 0)

    if with_cnt:
        ovec = jnp.ones((16,), jnp.float32)

        def _ones(i, _):
            ones_v[pl.ds(pl.multiple_of(i * 16, 16), 16)] = ovec
            return 0

        lax.fori_loop(0, CHUNK // 16, _ones, 0)

    for u in range(RPT // CHUNK):
        pltpu.sync_copy(rows_v, shared_agg.at[pl.ds(sid * RPT + u * CHUNK, CHUNK)])
    pltpu.sync_copy(zc_v, shared_cnt.at[pl.ds(sid * RPT, RPT)])
    plsc.subcore_barrier()

    my_cpw = jnp.where(cid == 0, CF, CS)
    bc = jnp.where(cid == 0, sid * CF, NS * CF + sid * CS)
    base = bc * CHUNK

    def _edge_chunk(j, _):
        @pl.when(j < my_cpw)
        def _():
            off = pl.multiple_of(base + j * CHUNK, 8)
            pltpu.sync_copy(src_hbm.at[pl.ds(off, CHUNK)], idx_s)
            pltpu.sync_copy(dst_hbm.at[pl.ds(off, CHUNK)], idx_d)
            pltpu.async_copy(h_hbm.at[idx_s], rows_v, sem).wait()
            pltpu.sync_copy(rows_v, shared_agg.at[idx_d], add=True)
            if with_cnt:
                pltpu.sync_copy(ones_v, shared_cnt.at[idx_d], add=True)
        return 0

    lax.fori_loop(0, CF, _edge_chunk, 0)
    plsc.subcore_barrier()

    row0 = sid * RPT
    pltpu.sync_copy(shared_agg.at[pl.ds(row0, RPT)],
                    agg_out.at[pl.ds(cid * N_PAD + row0, RPT)])
    if with_cnt:
        @pl.when(cid == 0)
        def _():
            pltpu.sync_copy(shared_cnt.at[pl.ds(row0, RPT)],
                            cnt_a.at[pl.ds(row0, RPT)])

        @pl.when(cid == 1)
        def _():
            pltpu.sync_copy(shared_cnt.at[pl.ds(row0, RPT)],
                            cnt_b.at[pl.ds(row0, RPT)])


def _make_sc(with_cnt):
    outs = [jax.ShapeDtypeStruct((NC * N_PAD, D), jnp.float32)]
    if with_cnt:
        outs += [jax.ShapeDtypeStruct((N_PAD,), jnp.float32)] * 2
    body = functools.partial(_sc_body, with_cnt)
    if not with_cnt:
        def body(h, srcv, dstv, agg, *rest):
            return _sc_body(False, h, srcv, dstv, agg, None, None, *rest)
    return pl.kernel(
        body,
        out_type=outs,
        mesh=_MESH,
        scratch_types=[
            pltpu.VMEM_SHARED((N_PAD, D), jnp.float32),
            pltpu.VMEM_SHARED((N_PAD,), jnp.float32),
            pltpu.VMEM((CHUNK, D), jnp.float32),
            pltpu.VMEM((CHUNK,), jnp.int32),
            pltpu.VMEM((CHUNK,), jnp.int32),
            pltpu.VMEM((CHUNK,), jnp.float32),
            pltpu.VMEM((RPT,), jnp.float32),
            pltpu.SemaphoreType.DMA,
        ],
    )


_sc_agg_cnt = _make_sc(True)
_sc_agg = _make_sc(False)

BN = 1024


def _tc_body(relu, agg0_ref, agg1_ref, ca_ref, cb_ref, h_ref, wn_ref, ws_ref,
             b_ref, out_ref):
    cnt = ca_ref[...] + cb_ref[...]
    inv = 1.0 / jnp.maximum(cnt, 1.0)
    agg = (agg0_ref[0] + agg1_ref[0]) * inv[:, None]
    acc = jnp.dot(agg, wn_ref[...], preferred_element_type=jnp.float32)
    acc += jnp.dot(h_ref[...], ws_ref[...], preferred_element_type=jnp.float32)
    acc += b_ref[...][None, :]
    if relu:
        acc = jnp.maximum(acc, 0.0)
    out_ref[...] = acc


def _make_tc(relu):
    grid = N_PAD // BN
    return pl.pallas_call(
        functools.partial(_tc_body, relu),
        grid=(grid,),
        in_specs=[
            pl.BlockSpec((1, BN, D), lambda i: (0, i, 0)),
            pl.BlockSpec((1, BN, D), lambda i: (1, i, 0)),
            pl.BlockSpec((BN,), lambda i: (i,)),
            pl.BlockSpec((BN,), lambda i: (i,)),
            pl.BlockSpec((BN, D), lambda i: (i, 0)),
            pl.BlockSpec((D, D), lambda i: (0, 0)),
            pl.BlockSpec((D, D), lambda i: (0, 0)),
            pl.BlockSpec((D,), lambda i: (0,)),
        ],
        out_specs=pl.BlockSpec((BN, D), lambda i: (i, 0)),
        out_shape=jax.ShapeDtypeStruct((N, D), jnp.float32),
    )


_tc_relu = _make_tc(True)
_tc_lin = _make_tc(False)


def kernel(x, edge_index, W_self_0, W_neigh_0, b_0, W_self_1, W_neigh_1, b_1,
           W_self_2, W_neigh_2, b_2):
    pad = E_PAD - E
    src = jnp.concatenate([edge_index[0], jnp.zeros((pad,), jnp.int32)])
    dst = jnp.concatenate([edge_index[1], jnp.full((pad,), N, jnp.int32)])
    agg_f, cnt_a, cnt_b = _sc_agg_cnt(x, src, dst)
    agg = agg_f.reshape(NC, N_PAD, D)
    h = _tc_relu(agg, agg, cnt_a, cnt_b, x, W_neigh_0, W_self_0, b_0)

    agg = _sc_agg(h, src, dst)[0].reshape(NC, N_PAD, D)
    h = _tc_relu(agg, agg, cnt_a, cnt_b, h, W_neigh_1, W_self_1, b_1)

    agg = _sc_agg(h, src, dst)[0].reshape(NC, N_PAD, D)
    return _tc_lin(agg, agg, cnt_a, cnt_b, h, W_neigh_2, W_self_2, b_2)


# speedup vs baseline: 1.4579x; 1.0312x over previous
import functools

import jax
import jax.numpy as jnp
from jax import lax
from jax.experimental import pallas as pl
from jax.experimental.pallas import tpu as pltpu
from jax.experimental.pallas import tpu_sc as plsc

N = 10000
D = 128
E = 320000

NC = 2
NS = 16
NW = NC * NS
CHUNK = 128
CPW = -(-E // (NW * CHUNK))
CF = 99
CS = 2 * CPW - CF
E_PAD = NW * CPW * CHUNK
N_PAD = 10240
RPT = N_PAD // NS

_MESH = plsc.VectorSubcoreMesh(core_axis_name="c", subcore_axis_name="s")


def _sc_body(with_cnt, h_hbm, src_hbm, dst_hbm, agg_out, cnt_a, cnt_b,
             shared_agg, shared_cnt, rows_v, idx_s, idx_d, ones_v, zc_v, sem):
    cid = lax.axis_index("c")
    sid = lax.axis_index("s")
    wid = sid * NC + cid

    zvec = jnp.zeros((16,), jnp.float32)

    def _zrow(i, _):
        for j in range(8):
            rows_v[i, pl.ds(j * 16, 16)] = zvec
        return 0

    lax.fori_loop(0, CHUNK, _zrow, 0)

    def _zc(i, _):
        zc_v[pl.ds(pl.multiple_of(i * 16, 16), 16)] = zvec
        return 0

    lax.fori_loop(0, RPT // 16, _zc, 0)

    if with_cnt:
        ovec = jnp.ones((16,), jnp.float32)

        def _ones(i, _):
            ones_v[pl.ds(pl.multiple_of(i * 16, 16), 16)] = ovec
            return 0

        lax.fori_loop(0, CHUNK // 16, _ones, 0)

    for u in range(RPT // CHUNK):
        pltpu.sync_copy(rows_v, shared_agg.at[pl.ds(sid * RPT + u * CHUNK, CHUNK)])
    pltpu.sync_copy(zc_v, shared_cnt.at[pl.ds(sid * RPT, RPT)])
    plsc.subcore_barrier()

    my_cpw = jnp.where(cid == 0, CF, CS)
    bc = jnp.where(cid == 0, sid * CF, NS * CF + sid * CS)
    base = bc * CHUNK

    def _edge_chunk(j, _):
        @pl.when(j < my_cpw)
        def _():
            off = pl.multiple_of(base + j * CHUNK, 8)
            pltpu.sync_copy(src_hbm.at[pl.ds(off, CHUNK)], idx_s)
            pltpu.sync_copy(dst_hbm.at[pl.ds(off, CHUNK)], idx_d)
            pltpu.async_copy(h_hbm.at[idx_s], rows_v, sem).wait()
            pltpu.sync_copy(rows_v, shared_agg.at[idx_d], add=True)
            if with_cnt:
                pltpu.sync_copy(ones_v, shared_cnt.at[idx_d], add=True)
        return 0

    lax.fori_loop(0, CF, _edge_chunk, 0)
    plsc.subcore_barrier()

    row0 = sid * RPT
    pltpu.sync_copy(shared_agg.at[pl.ds(row0, RPT)],
                    agg_out.at[pl.ds(cid * N_PAD + row0, RPT)])
    if with_cnt:
        @pl.when(cid == 0)
        def _():
            pltpu.sync_copy(shared_cnt.at[pl.ds(row0, RPT)],
                            cnt_a.at[pl.ds(row0, RPT)])

        @pl.when(cid == 1)
        def _():
            pltpu.sync_copy(shared_cnt.at[pl.ds(row0, RPT)],
                            cnt_b.at[pl.ds(row0, RPT)])


def _make_sc(with_cnt):
    outs = [jax.ShapeDtypeStruct((NC * N_PAD, D), jnp.float32)]
    if with_cnt:
        outs += [jax.ShapeDtypeStruct((N_PAD,), jnp.float32)] * 2
    body = functools.partial(_sc_body, with_cnt)
    if not with_cnt:
        def body(h, srcv, dstv, agg, *rest):
            return _sc_body(False, h, srcv, dstv, agg, None, None, *rest)
    return pl.kernel(
        body,
        out_type=outs,
        mesh=_MESH,
        scratch_types=[
            pltpu.VMEM_SHARED((N_PAD, D), jnp.float32),
            pltpu.VMEM_SHARED((N_PAD,), jnp.float32),
            pltpu.VMEM((CHUNK, D), jnp.float32),
            pltpu.VMEM((CHUNK,), jnp.int32),
            pltpu.VMEM((CHUNK,), jnp.int32),
            pltpu.VMEM((CHUNK,), jnp.float32),
            pltpu.VMEM((RPT,), jnp.float32),
            pltpu.SemaphoreType.DMA,
        ],
    )


_sc_agg_cnt = _make_sc(True)
_sc_agg = _make_sc(False)

BN = 1024


def _tc_body(relu, agg0_ref, agg1_ref, ca_ref, cb_ref, h_ref, wn_ref, ws_ref,
             b_ref, out_ref):
    cnt = ca_ref[...] + cb_ref[...]
    inv = 1.0 / jnp.maximum(cnt, 1.0)
    agg = (agg0_ref[0] + agg1_ref[0]) * inv[:, None]
    acc = jnp.dot(agg, wn_ref[...], preferred_element_type=jnp.float32)
    acc += jnp.dot(h_ref[...], ws_ref[...], preferred_element_type=jnp.float32)
    acc += b_ref[...][None, :]
    if relu:
        acc = jnp.maximum(acc, 0.0)
    out_ref[...] = acc


def _make_tc(relu):
    grid = N_PAD // BN
    return pl.pallas_call(
        functools.partial(_tc_body, relu),
        grid=(grid,),
        in_specs=[
            pl.BlockSpec((1, BN, D), lambda i: (0, i, 0)),
            pl.BlockSpec((1, BN, D), lambda i: (1, i, 0)),
            pl.BlockSpec((BN,), lambda i: (i,)),
            pl.BlockSpec((BN,), lambda i: (i,)),
            pl.BlockSpec((BN, D), lambda i: (i, 0)),
            pl.BlockSpec((D, D), lambda i: (0, 0)),
            pl.BlockSpec((D, D), lambda i: (0, 0)),
            pl.BlockSpec((D,), lambda i: (0,)),
        ],
        out_specs=pl.BlockSpec((BN, D), lambda i: (i, 0)),
        out_shape=jax.ShapeDtypeStruct((N, D), jnp.float32),
    )


_tc_relu = _make_tc(True)
_tc_lin = _make_tc(False)


def kernel(x, edge_index, W_self_0, W_neigh_0, b_0, W_self_1, W_neigh_1, b_1,
           W_self_2, W_neigh_2, b_2):
    pad = E_PAD - E
    src = jnp.concatenate([edge_index[0], jnp.zeros((pad,), jnp.int32)])
    dst = jnp.concatenate([edge_index[1], jnp.full((pad,), N, jnp.int32)])
    agg_f, cnt_a, cnt_b = _sc_agg_cnt(x, src, dst)
    agg = agg_f.reshape(NC, N_PAD, D)
    h = _tc_relu(agg, agg, cnt_a, cnt_b, x, W_neigh_0, W_self_0, b_0)

    agg = _sc_agg(h, src, dst)[0].reshape(NC, N_PAD, D)
    h = _tc_relu(agg, agg, cnt_a, cnt_b, h, W_neigh_1, W_self_1, b_1)

    agg = _sc_agg(h, src, dst)[0].reshape(NC, N_PAD, D)
    return _tc_lin(agg, agg, cnt_a, cnt_b, h, W_neigh_2, W_self_2, b_2)


# speedup vs baseline: 1.4755x; 1.0121x over previous
import functools

import jax
import jax.numpy as jnp
from jax import lax
from jax.experimental import pallas as pl
from jax.experimental.pallas import tpu as pltpu
from jax.experimental.pallas import tpu_sc as plsc

N = 10000
D = 128
E = 320000

NC = 2
NS = 16
NW = NC * NS
CHUNK = 128
CPW = -(-E // (NW * CHUNK))
CF = 103
CS = 2 * CPW - CF
E_PAD = NW * CPW * CHUNK
N_PAD = 10240
RPT = N_PAD // NS

_MESH = plsc.VectorSubcoreMesh(core_axis_name="c", subcore_axis_name="s")


def _sc_body(with_cnt, h_hbm, src_hbm, dst_hbm, agg_out, cnt_a, cnt_b,
             shared_agg, shared_cnt, rows_v, idx_s, idx_d, ones_v, zc_v, sem):
    cid = lax.axis_index("c")
    sid = lax.axis_index("s")
    wid = sid * NC + cid

    zvec = jnp.zeros((16,), jnp.float32)

    def _zrow(i, _):
        for j in range(8):
            rows_v[i, pl.ds(j * 16, 16)] = zvec
        return 0

    lax.fori_loop(0, CHUNK, _zrow, 0)

    def _zc(i, _):
        zc_v[pl.ds(pl.multiple_of(i * 16, 16), 16)] = zvec
        return 0

    lax.fori_loop(0, RPT // 16, _zc, 0)

    if with_cnt:
        ovec = jnp.ones((16,), jnp.float32)

        def _ones(i, _):
            ones_v[pl.ds(pl.multiple_of(i * 16, 16), 16)] = ovec
            return 0

        lax.fori_loop(0, CHUNK // 16, _ones, 0)

    for u in range(RPT // CHUNK):
        pltpu.sync_copy(rows_v, shared_agg.at[pl.ds(sid * RPT + u * CHUNK, CHUNK)])
    pltpu.sync_copy(zc_v, shared_cnt.at[pl.ds(sid * RPT, RPT)])
    plsc.subcore_barrier()

    my_cpw = jnp.where(cid == 0, CF, CS)
    bc = jnp.where(cid == 0, sid * CF, NS * CF + sid * CS)
    base = bc * CHUNK

    def _edge_chunk(j, _):
        @pl.when(j < my_cpw)
        def _():
            off = pl.multiple_of(base + j * CHUNK, 8)
            pltpu.sync_copy(src_hbm.at[pl.ds(off, CHUNK)], idx_s)
            pltpu.sync_copy(dst_hbm.at[pl.ds(off, CHUNK)], idx_d)
            pltpu.async_copy(h_hbm.at[idx_s], rows_v, sem).wait()
            pltpu.sync_copy(rows_v, shared_agg.at[idx_d], add=True)
            if with_cnt:
                pltpu.sync_copy(ones_v, shared_cnt.at[idx_d], add=True)
        return 0

    lax.fori_loop(0, CF, _edge_chunk, 0)
    plsc.subcore_barrier()

    row0 = sid * RPT
    pltpu.sync_copy(shared_agg.at[pl.ds(row0, RPT)],
                    agg_out.at[pl.ds(cid * N_PAD + row0, RPT)])
    if with_cnt:
        @pl.when(cid == 0)
        def _():
            pltpu.sync_copy(shared_cnt.at[pl.ds(row0, RPT)],
                            cnt_a.at[pl.ds(row0, RPT)])

        @pl.when(cid == 1)
        def _():
            pltpu.sync_copy(shared_cnt.at[pl.ds(row0, RPT)],
                            cnt_b.at[pl.ds(row0, RPT)])


def _make_sc(with_cnt):
    outs = [jax.ShapeDtypeStruct((NC * N_PAD, D), jnp.float32)]
    if with_cnt:
        outs += [jax.ShapeDtypeStruct((N_PAD,), jnp.float32)] * 2
    body = functools.partial(_sc_body, with_cnt)
    if not with_cnt:
        def body(h, srcv, dstv, agg, *rest):
            return _sc_body(False, h, srcv, dstv, agg, None, None, *rest)
    return pl.kernel(
        body,
        out_type=outs,
        mesh=_MESH,
        scratch_types=[
            pltpu.VMEM_SHARED((N_PAD, D), jnp.float32),
            pltpu.VMEM_SHARED((N_PAD,), jnp.float32),
            pltpu.VMEM((CHUNK, D), jnp.float32),
            pltpu.VMEM((CHUNK,), jnp.int32),
            pltpu.VMEM((CHUNK,), jnp.int32),
            pltpu.VMEM((CHUNK,), jnp.float32),
            pltpu.VMEM((RPT,), jnp.float32),
            pltpu.SemaphoreType.DMA,
        ],
    )


_sc_agg_cnt = _make_sc(True)
_sc_agg = _make_sc(False)

BN = 1024


def _tc_body(relu, agg0_ref, agg1_ref, ca_ref, cb_ref, h_ref, wn_ref, ws_ref,
             b_ref, out_ref):
    cnt = ca_ref[...] + cb_ref[...]
    inv = 1.0 / jnp.maximum(cnt, 1.0)
    agg = (agg0_ref[0] + agg1_ref[0]) * inv[:, None]
    acc = jnp.dot(agg, wn_ref[...], preferred_element_type=jnp.float32)
    acc += jnp.dot(h_ref[...], ws_ref[...], preferred_element_type=jnp.float32)
    acc += b_ref[...][None, :]
    if relu:
        acc = jnp.maximum(acc, 0.0)
    out_ref[...] = acc


def _make_tc(relu):
    grid = N_PAD // BN
    return pl.pallas_call(
        functools.partial(_tc_body, relu),
        grid=(grid,),
        in_specs=[
            pl.BlockSpec((1, BN, D), lambda i: (0, i, 0)),
            pl.BlockSpec((1, BN, D), lambda i: (1, i, 0)),
            pl.BlockSpec((BN,), lambda i: (i,)),
            pl.BlockSpec((BN,), lambda i: (i,)),
            pl.BlockSpec((BN, D), lambda i: (i, 0)),
            pl.BlockSpec((D, D), lambda i: (0, 0)),
            pl.BlockSpec((D, D), lambda i: (0, 0)),
            pl.BlockSpec((D,), lambda i: (0,)),
        ],
        out_specs=pl.BlockSpec((BN, D), lambda i: (i, 0)),
        out_shape=jax.ShapeDtypeStruct((N, D), jnp.float32),
    )


_tc_relu = _make_tc(True)
_tc_lin = _make_tc(False)


def kernel(x, edge_index, W_self_0, W_neigh_0, b_0, W_self_1, W_neigh_1, b_1,
           W_self_2, W_neigh_2, b_2):
    pad = E_PAD - E
    src = jnp.concatenate([edge_index[0], jnp.zeros((pad,), jnp.int32)])
    dst = jnp.concatenate([edge_index[1], jnp.full((pad,), N, jnp.int32)])
    agg_f, cnt_a, cnt_b = _sc_agg_cnt(x, src, dst)
    agg = agg_f.reshape(NC, N_PAD, D)
    h = _tc_relu(agg, agg, cnt_a, cnt_b, x, W_neigh_0, W_self_0, b_0)

    agg = _sc_agg(h, src, dst)[0].reshape(NC, N_PAD, D)
    h = _tc_relu(agg, agg, cnt_a, cnt_b, h, W_neigh_1, W_self_1, b_1)

    agg = _sc_agg(h, src, dst)[0].reshape(NC, N_PAD, D)
    return _tc_lin(agg, agg, cnt_a, cnt_b, h, W_neigh_2, W_self_2, b_2)
